# baseline (speedup 1.0000x reference)
import jax
import jax.numpy as jnp
from jax.experimental import pallas as pl

RB = 16


def _body(gu_ref, gi_ref, xui_ref):
    p = gu_ref[...] * gi_ref[...]
    xui_ref[...] = jnp.sum(p.reshape(RB, 128, 64), axis=-1)


def kernel(gu, gi):
    B, K = gu.shape
    W = B * K // 128
    gu2 = gu.reshape(128, W)
    gi2 = gi.reshape(128, W)
    grid = (128 // RB,)
    xui = pl.pallas_call(
        _body,
        grid=grid,
        in_specs=[
            pl.BlockSpec((RB, W), lambda i: (i, 0)),
            pl.BlockSpec((RB, W), lambda i: (i, 0)),
        ],
        out_specs=pl.BlockSpec((RB, 128), lambda i: (i, 0)),
        out_shape=jax.ShapeDtypeStruct((128, 128), gu.dtype),
    )(gu2, gi2)
    return (xui.reshape(B), gu, gi)


# speedup vs baseline: 3.1761x; 3.1761x over previous
import jax
import jax.numpy as jnp
from jax.experimental import pallas as pl

CB = 2048


def _body(gu_ref, gi_ref, xui_ref):
    xui_ref[...] = jnp.sum(gu_ref[...] * gi_ref[...], axis=0)


def kernel(gu, gi):
    B, K = gu.shape
    gut = gu.T
    git = gi.T
    grid = (B // CB,)
    xui = pl.pallas_call(
        _body,
        grid=grid,
        in_specs=[
            pl.BlockSpec((K, CB), lambda i: (0, i)),
            pl.BlockSpec((K, CB), lambda i: (0, i)),
        ],
        out_specs=pl.BlockSpec((CB,), lambda i: (i,)),
        out_shape=jax.ShapeDtypeStruct((B,), gu.dtype),
    )(gut, git)
    return (xui, gu, gi)
